# Initial kernel scaffold; baseline (speedup 1.0000x reference)
import jax
import jax.numpy as jnp
from jax.experimental import pallas as pl


def kernel(z_e, emb):
    raise NotImplementedError("write your pallas kernel here")


# speedup vs baseline: 1.0034x; 1.0034x over previous
import jax
import jax.numpy as jnp
from jax.experimental import pallas as pl

NUM_EMB = 1024
DIM = 64
BETA = 0.25


def _vq_block(z_ref, emb_ref, quant_ref, idx_ref, loss_ref):
    i = pl.program_id(0)
    nblk = pl.num_programs(0)
    z = z_ref[...]
    emb = emb_ref[...]

    sz = jnp.sum(z * z, axis=1, keepdims=True)
    se = jnp.sum(emb * emb, axis=1)
    mm = jax.lax.dot_general(z, emb, (((1,), (1,)), ((), ())),
                             preferred_element_type=jnp.float32)
    d = (sz + se[None, :]) - 2.0 * mm

    dmin = jnp.min(d, axis=1, keepdims=True)
    iota = jax.lax.broadcasted_iota(jnp.int32, d.shape, 1)
    idxm = jnp.where(d == dmin, iota, NUM_EMB)
    idx = jnp.min(idxm, axis=1)
    idx_ref[...] = idx

    onehot = (iota == idx[:, None]).astype(jnp.float32)
    q = jax.lax.dot_general(onehot, emb, (((1,), (0,)), ((), ())),
                            precision=jax.lax.Precision.HIGHEST,
                            preferred_element_type=jnp.float32)

    quant = z + (q - z)
    quant_ref[...] = quant

    diff = quant - z
    bsum = jnp.sum(diff * diff, axis=(0, 1), keepdims=True)

    @pl.when(i == 0)
    def _init():
        loss_ref[...] = bsum

    @pl.when(i != 0)
    def _acc():
        loss_ref[...] += bsum


def kernel(z_e, emb):
    M = z_e.size // DIM
    z_flat = z_e.reshape(M, DIM)
    BM = 512
    grid = M // BM
    quant, idx, losssum = pl.pallas_call(
        _vq_block,
        grid=(grid,),
        in_specs=[pl.BlockSpec((BM, DIM), lambda i: (i, 0)),
                  pl.BlockSpec((NUM_EMB, DIM), lambda i: (0, 0))],
        out_specs=[pl.BlockSpec((BM, DIM), lambda i: (i, 0)),
                   pl.BlockSpec((BM,), lambda i: (i,)),
                   pl.BlockSpec((1, 1), lambda i: (0, 0))],
        out_shape=[jax.ShapeDtypeStruct((M, DIM), jnp.float32),
                   jax.ShapeDtypeStruct((M,), jnp.int32),
                   jax.ShapeDtypeStruct((1, 1), jnp.float32)],
    )(z_flat, emb)

    mean_sq = losssum[0, 0] / jnp.float32(M * DIM)
    loss = mean_sq + BETA * mean_sq
    quantized = quant.reshape(z_e.shape)
    encoding_indices = idx[:, None]
    return (loss, quantized, encoding_indices)


# speedup vs baseline: 1.2443x; 1.2400x over previous
import functools

import jax
import jax.numpy as jnp
from jax import lax
from jax.experimental import pallas as pl
from jax.experimental.pallas import tpu as pltpu
from jax.experimental.pallas import tpu_sc as plsc

NUM_EMB = 1024
DIM = 64
BETA = 0.25

M_TOTAL = 16384
BM = 512
_NW = 32
_ROWS_PER_TILE = M_TOTAL // _NW


def _vq_block(z_ref, emb_ref, idx_ref, loss_ref):
    i = pl.program_id(0)
    z = z_ref[...]
    emb = emb_ref[...]

    sz = jnp.sum(z * z, axis=1, keepdims=True)
    se = jnp.sum(emb * emb, axis=1)
    mm = jax.lax.dot_general(z, emb, (((1,), (1,)), ((), ())),
                             preferred_element_type=jnp.float32)
    d = (sz + se[None, :]) - 2.0 * mm

    dmin = jnp.min(d, axis=1, keepdims=True)
    iota = jax.lax.broadcasted_iota(jnp.int32, d.shape, 1)
    idxm = jnp.where(d == dmin, iota, NUM_EMB)
    idx_ref[...] = jnp.min(idxm, axis=1)

    bsum = jnp.sum(dmin, axis=(0, 1), keepdims=True)

    @pl.when(i == 0)
    def _init():
        loss_ref[...] = bsum

    @pl.when(i != 0)
    def _acc():
        loss_ref[...] += bsum


_sc_mesh = plsc.VectorSubcoreMesh(core_axis_name="c", subcore_axis_name="s")


_IDX_CHUNK = 128


@functools.partial(
    pl.kernel,
    out_type=jax.ShapeDtypeStruct((M_TOTAL, 2 * DIM), jnp.float32),
    mesh=_sc_mesh,
    scratch_types=[
        pltpu.VMEM((_ROWS_PER_TILE,), jnp.int32),
        pltpu.VMEM((_ROWS_PER_TILE, 2 * DIM), jnp.float32),
        pltpu.SemaphoreType.DMA,
    ],
)
def _sc_gather(emb_hbm, idx_hbm, out_hbm, idx_v, rows_v, sem):
    wid = lax.axis_index("s") * 2 + lax.axis_index("c")
    base = wid * _ROWS_PER_TILE
    pltpu.sync_copy(idx_hbm.at[pl.ds(base, _ROWS_PER_TILE)], idx_v)
    copies = []
    for j in range(_ROWS_PER_TILE // _IDX_CHUNK):
        copies.append(pltpu.async_copy(
            emb_hbm.at[idx_v.at[pl.ds(j * _IDX_CHUNK, _IDX_CHUNK)]],
            rows_v.at[pl.ds(j * _IDX_CHUNK, _IDX_CHUNK)],
            sem))
    for c in copies:
        c.wait()
    pltpu.sync_copy(rows_v, out_hbm.at[pl.ds(base, _ROWS_PER_TILE)])


def kernel(z_e, emb):
    z_flat = z_e.reshape(M_TOTAL, DIM)
    grid = M_TOTAL // BM
    idx, losssum = pl.pallas_call(
        _vq_block,
        grid=(grid,),
        in_specs=[pl.BlockSpec((BM, DIM), lambda i: (i, 0)),
                  pl.BlockSpec((NUM_EMB, DIM), lambda i: (0, 0))],
        out_specs=[pl.BlockSpec((BM,), lambda i: (i,)),
                   pl.BlockSpec((1, 1), lambda i: (0, 0))],
        out_shape=[jax.ShapeDtypeStruct((M_TOTAL,), jnp.int32),
                   jax.ShapeDtypeStruct((1, 1), jnp.float32)],
    )(z_flat, emb)

    emb_padded = jnp.pad(emb, ((0, 0), (0, 2 * DIM - DIM)))
    quant = _sc_gather(emb_padded, idx)[:, :DIM]

    mean_sq = losssum[0, 0] / jnp.float32(M_TOTAL * DIM)
    loss = mean_sq + BETA * mean_sq
    quantized = quant.reshape(z_e.shape)
    encoding_indices = idx[:, None]
    return (loss, quantized, encoding_indices)


# speedup vs baseline: 1.3161x; 1.0577x over previous
import functools

import jax
import jax.numpy as jnp
from jax import lax
from jax.experimental import pallas as pl
from jax.experimental.pallas import tpu as pltpu
from jax.experimental.pallas import tpu_sc as plsc

NUM_EMB = 1024
DIM = 64
BETA = 0.25

M_TOTAL = 16384
BM = 512
_NW = 32
_ROWS_PER_TILE = M_TOTAL // _NW
_IDX_CHUNK = 128


def _vq_block(z_ref, emb_ref, idx_ref, loss_ref):
    i = pl.program_id(0)
    z = z_ref[...]
    emb = emb_ref[...]

    sz = jnp.sum(z * z, axis=1, keepdims=True)
    se = jnp.sum(emb * emb, axis=1)
    mm = jax.lax.dot_general(z, emb, (((1,), (1,)), ((), ())),
                             preferred_element_type=jnp.float32)
    d = (sz + se[None, :]) - 2.0 * mm

    dmin = jnp.min(d, axis=1, keepdims=True)
    iota = jax.lax.broadcasted_iota(jnp.int32, d.shape, 1).astype(jnp.float32)
    idxm = jnp.where(d == dmin, iota, jnp.float32(NUM_EMB))
    idx_ref[...] = jnp.min(idxm, axis=1).astype(jnp.int32)

    bsum = jnp.sum(dmin, axis=(0, 1), keepdims=True)

    @pl.when(i == 0)
    def _init():
        loss_ref[...] = bsum

    @pl.when(i != 0)
    def _acc():
        loss_ref[...] += bsum


_sc_mesh = plsc.VectorSubcoreMesh(core_axis_name="c", subcore_axis_name="s")


@functools.partial(
    pl.kernel,
    out_type=jax.ShapeDtypeStruct((M_TOTAL // 2, 2 * DIM), jnp.float32),
    mesh=_sc_mesh,
    scratch_types=[
        pltpu.VMEM((_ROWS_PER_TILE,), jnp.int32),
        pltpu.VMEM((_ROWS_PER_TILE, 2 * DIM), jnp.float32),
        pltpu.VMEM((_ROWS_PER_TILE // 2, 2 * DIM), jnp.float32),
        pltpu.SemaphoreType.DMA,
    ],
)
def _sc_gather(emb_hbm, idx_hbm, out_hbm, idx_v, rows_v, packed_v, sem):
    wid = lax.axis_index("s") * 2 + lax.axis_index("c")
    base = wid * _ROWS_PER_TILE
    pltpu.sync_copy(idx_hbm.at[pl.ds(base, _ROWS_PER_TILE)], idx_v)
    copies = []
    for j in range(_ROWS_PER_TILE // _IDX_CHUNK):
        copies.append(pltpu.async_copy(
            emb_hbm.at[idx_v.at[pl.ds(j * _IDX_CHUNK, _IDX_CHUNK)]],
            rows_v.at[pl.ds(j * _IDX_CHUNK, _IDX_CHUNK)],
            sem))
    for c in copies:
        c.wait()

    def _pack(r, carry):
        half = (r % 2) * DIM
        for j in range(DIM // 16):
            v = rows_v.at[r][pl.ds(j * 16, 16)]
            packed_v.at[r // 2][pl.ds(half + j * 16, 16)] = v
        return carry

    lax.fori_loop(0, _ROWS_PER_TILE, _pack, 0, unroll=2)

    pltpu.sync_copy(packed_v,
                    out_hbm.at[pl.ds(wid * (_ROWS_PER_TILE // 2),
                                     _ROWS_PER_TILE // 2)])


def kernel(z_e, emb):
    z_flat = z_e.reshape(M_TOTAL, DIM)
    grid = M_TOTAL // BM
    idx, losssum = pl.pallas_call(
        _vq_block,
        grid=(grid,),
        in_specs=[pl.BlockSpec((BM, DIM), lambda i: (i, 0)),
                  pl.BlockSpec((NUM_EMB, DIM), lambda i: (0, 0))],
        out_specs=[pl.BlockSpec((BM,), lambda i: (i,)),
                   pl.BlockSpec((1, 1), lambda i: (0, 0))],
        out_shape=[jax.ShapeDtypeStruct((M_TOTAL,), jnp.int32),
                   jax.ShapeDtypeStruct((1, 1), jnp.float32)],
    )(z_flat, emb)

    emb_padded = jnp.pad(emb, ((0, 0), (0, 2 * DIM - DIM)))
    quant = _sc_gather(emb_padded, idx)

    mean_sq = losssum[0, 0] / jnp.float32(M_TOTAL * DIM)
    loss = mean_sq + BETA * mean_sq
    quantized = quant.reshape(z_e.shape)
    encoding_indices = idx[:, None]
    return (loss, quantized, encoding_indices)


# speedup vs baseline: 1.9304x; 1.4668x over previous
import functools

import jax
import jax.numpy as jnp
from jax import lax
from jax.experimental import pallas as pl
from jax.experimental.pallas import tpu as pltpu
from jax.experimental.pallas import tpu_sc as plsc

NUM_EMB = 1024
DIM = 64
BETA = 0.25

M_TOTAL = 16384
BM = 1024
_NB = 16
_NW = 32
_ROWS_PER_TILE = M_TOTAL // _NW
_SG_PER_TILE = _ROWS_PER_TILE // DIM


def _vq_block(z_ref, emb_ref, idx_ref, idxsc_ref, loss_ref):
    i = pl.program_id(0)
    znhwc = z_ref[...]
    emb = emb_ref[...]

    a3 = znhwc.reshape(_NB, DIM, DIM)
    zt = jnp.transpose(a3, (0, 2, 1))
    z = zt.reshape(BM, DIM)

    sz = jnp.sum(z * z, axis=1, keepdims=True)
    se = jnp.sum(emb * emb, axis=1)
    mm = jax.lax.dot_general(z, emb, (((1,), (1,)), ((), ())),
                             preferred_element_type=jnp.float32)
    d = (sz + se[None, :]) - 2.0 * mm

    dmin = jnp.min(d, axis=1, keepdims=True)
    iota = jax.lax.broadcasted_iota(jnp.int32, d.shape, 1).astype(jnp.float32)
    idxm = jnp.where(d == dmin, iota, jnp.float32(NUM_EMB))
    idx_f = jnp.min(idxm, axis=1)
    idxsc_ref[...] = idx_f.astype(jnp.int32)
    idx_ref[...] = jnp.transpose(idx_f.reshape(_NB, DIM)).astype(jnp.int32)

    bsum = jnp.sum(dmin, axis=(0, 1), keepdims=True)

    @pl.when(i == 0)
    def _init():
        loss_ref[...] = bsum

    @pl.when(i != 0)
    def _acc():
        loss_ref[...] += bsum


@functools.cache
def _make_sc_gather():
    mesh = plsc.VectorSubcoreMesh(core_axis_name="c", subcore_axis_name="s")
    return functools.partial(
        pl.kernel,
        out_type=jax.ShapeDtypeStruct((M_TOTAL // 2, 2 * DIM), jnp.float32),
        mesh=mesh,
        scratch_types=[
            pltpu.VMEM((_ROWS_PER_TILE,), jnp.int32),
            pltpu.VMEM((DIM * NUM_EMB,), jnp.float32),
            pltpu.VMEM((_ROWS_PER_TILE // 2, 2 * DIM), jnp.float32),
            pltpu.SemaphoreType.DMA,
        ],
        compiler_params=pltpu.CompilerParams(needs_layout_passes=False),
    )(_sc_gather_body)


def _sc_gather_body(embt_hbm, idxsc_hbm, out_hbm, idx_v, embt_v, packed_v, sem):
    wid = lax.axis_index("s") * 2 + lax.axis_index("c")
    base = wid * _ROWS_PER_TILE
    pltpu.sync_copy(idxsc_hbm.at[pl.ds(base, _ROWS_PER_TILE)], idx_v)
    pltpu.async_copy(embt_hbm, embt_v, sem).wait()

    for s in range(_SG_PER_TILE):
        iq = [idx_v[pl.ds(s * DIM + q * 16, 16)] for q in range(DIM // 16)]

        def _row(j, carry):
            off = (s * DIM + j) * DIM
            row = off // 128
            col = off % 128
            for q in range(DIM // 16):
                vals = plsc.load_gather(embt_v, [iq[q] + j * NUM_EMB])
                packed_v.at[row][pl.ds(col + q * 16, 16)] = vals
            return carry

        lax.fori_loop(0, DIM, _row, 0)

    pltpu.sync_copy(packed_v,
                    out_hbm.at[pl.ds(wid * (_ROWS_PER_TILE // 2),
                                     _ROWS_PER_TILE // 2)])


def kernel(z_e, emb):
    z_nhwc = jnp.transpose(z_e, (0, 2, 3, 1)).reshape(M_TOTAL, DIM)
    grid = M_TOTAL // BM
    idx_log, idx_sc, losssum = pl.pallas_call(
        _vq_block,
        grid=(grid,),
        in_specs=[pl.BlockSpec((BM, DIM), lambda i: (i, 0)),
                  pl.BlockSpec((NUM_EMB, DIM), lambda i: (0, 0))],
        out_specs=[pl.BlockSpec((DIM, _NB), lambda i: (i, 0)),
                   pl.BlockSpec((BM,), lambda i: (i,)),
                   pl.BlockSpec((1, 1), lambda i: (0, 0))],
        out_shape=[jax.ShapeDtypeStruct((M_TOTAL // _NB, _NB), jnp.int32),
                   jax.ShapeDtypeStruct((M_TOTAL,), jnp.int32),
                   jax.ShapeDtypeStruct((1, 1), jnp.float32)],
    )(z_nhwc, emb)

    embt_flat = jnp.transpose(emb).reshape(DIM * NUM_EMB)
    quant_packed = _make_sc_gather()(embt_flat, idx_sc)

    mean_sq = losssum[0, 0] / jnp.float32(M_TOTAL * DIM)
    loss = mean_sq + BETA * mean_sq
    quantized = jnp.transpose(
        quant_packed.reshape(16, 32, 32, DIM), (0, 3, 1, 2))
    encoding_indices = idx_log.reshape(M_TOTAL)[:, None]
    return (loss, quantized, encoding_indices)


# speedup vs baseline: 1.9555x; 1.0130x over previous
import functools

import jax
import jax.numpy as jnp
from jax import lax
from jax.experimental import pallas as pl
from jax.experimental.pallas import tpu as pltpu
from jax.experimental.pallas import tpu_sc as plsc

NUM_EMB = 1024
DIM = 64
BETA = 0.25

M_TOTAL = 16384
BM = 1024
_NB = 16
_NW = 32
_ROWS_PER_TILE = M_TOTAL // _NW
_SG_PER_TILE = _ROWS_PER_TILE // DIM


def _vq_block(z_ref, emb_ref, idx_ref, idxsc_ref, loss_ref):
    i = pl.program_id(0)
    znhwc = z_ref[...]
    emb = emb_ref[...]

    a3 = znhwc.reshape(_NB, DIM, DIM)
    zt = jnp.transpose(a3, (0, 2, 1))
    z = zt.reshape(BM, DIM)

    sz = jnp.sum(z * z, axis=1, keepdims=True)
    se = jnp.sum(emb * emb, axis=1)
    mm = jax.lax.dot_general(z, emb, (((1,), (1,)), ((), ())),
                             preferred_element_type=jnp.float32)
    d = (sz + se[None, :]) - 2.0 * mm

    dmin = jnp.min(d, axis=1, keepdims=True)
    iota = jax.lax.broadcasted_iota(jnp.int32, d.shape, 1).astype(jnp.float32)
    idxm = jnp.where(d == dmin, iota, jnp.float32(NUM_EMB))
    idx_f = jnp.min(idxm, axis=1)
    idxsc_ref[...] = idx_f.astype(jnp.int32)
    idx_ref[...] = jnp.transpose(idx_f.reshape(_NB, DIM)).astype(jnp.int32)

    bsum = jnp.sum(dmin, axis=(0, 1), keepdims=True)

    @pl.when(i == 0)
    def _init():
        loss_ref[...] = bsum

    @pl.when(i != 0)
    def _acc():
        loss_ref[...] += bsum


@functools.cache
def _make_sc_gather():
    mesh = plsc.VectorSubcoreMesh(core_axis_name="c", subcore_axis_name="s")
    return functools.partial(
        pl.kernel,
        out_type=jax.ShapeDtypeStruct((M_TOTAL // 2, 2 * DIM), jnp.float32),
        mesh=mesh,
        scratch_types=[
            pltpu.VMEM((_ROWS_PER_TILE,), jnp.int32),
            pltpu.VMEM((DIM * NUM_EMB,), jnp.float32),
            pltpu.VMEM((_ROWS_PER_TILE // 2, 2 * DIM), jnp.float32),
            pltpu.SemaphoreType.DMA,
        ],
        compiler_params=pltpu.CompilerParams(needs_layout_passes=False),
    )(_sc_gather_body)


def _sc_gather_body(embt_hbm, idxsc_hbm, out_hbm, idx_v, embt_v, packed_v, sem):
    wid = lax.axis_index("s") * 2 + lax.axis_index("c")
    base = wid * _ROWS_PER_TILE
    emb_copy = pltpu.async_copy(embt_hbm, embt_v, sem)
    pltpu.sync_copy(idxsc_hbm.at[pl.ds(base, _ROWS_PER_TILE)], idx_v)
    emb_copy.wait()

    out_copies = []
    for s in range(_SG_PER_TILE):
        iq = [idx_v[pl.ds(s * DIM + q * 16, 16)] for q in range(DIM // 16)]

        def _row(j, carry):
            off = (s * DIM + j) * DIM
            row = off // 128
            col = off % 128
            for q in range(DIM // 16):
                vals = plsc.load_gather(embt_v, [iq[q] + j * NUM_EMB])
                packed_v.at[row][pl.ds(col + q * 16, 16)] = vals
            return carry

        lax.fori_loop(0, DIM, _row, 0, unroll=4)

        rows_per_sg = DIM // 2
        out_copies.append(pltpu.async_copy(
            packed_v.at[pl.ds(s * rows_per_sg, rows_per_sg)],
            out_hbm.at[pl.ds(wid * (_ROWS_PER_TILE // 2) + s * rows_per_sg,
                             rows_per_sg)],
            sem))
    for c in out_copies:
        c.wait()


def kernel(z_e, emb):
    z_nhwc = jnp.transpose(z_e, (0, 2, 3, 1)).reshape(M_TOTAL, DIM)
    grid = M_TOTAL // BM
    idx_log, idx_sc, losssum = pl.pallas_call(
        _vq_block,
        grid=(grid,),
        in_specs=[pl.BlockSpec((BM, DIM), lambda i: (i, 0)),
                  pl.BlockSpec((NUM_EMB, DIM), lambda i: (0, 0))],
        out_specs=[pl.BlockSpec((DIM, _NB), lambda i: (i, 0)),
                   pl.BlockSpec((BM,), lambda i: (i,)),
                   pl.BlockSpec((1, 1), lambda i: (0, 0))],
        out_shape=[jax.ShapeDtypeStruct((M_TOTAL // _NB, _NB), jnp.int32),
                   jax.ShapeDtypeStruct((M_TOTAL,), jnp.int32),
                   jax.ShapeDtypeStruct((1, 1), jnp.float32)],
    )(z_nhwc, emb)

    embt_flat = jnp.transpose(emb).reshape(DIM * NUM_EMB)
    quant_packed = _make_sc_gather()(embt_flat, idx_sc)

    mean_sq = losssum[0, 0] / jnp.float32(M_TOTAL * DIM)
    loss = mean_sq + BETA * mean_sq
    quantized = jnp.transpose(
        quant_packed.reshape(16, 32, 32, DIM), (0, 3, 1, 2))
    encoding_indices = idx_log.reshape(M_TOTAL)[:, None]
    return (loss, quantized, encoding_indices)


# speedup vs baseline: 1.9889x; 1.0171x over previous
import functools

import jax
import jax.numpy as jnp
from jax import lax
from jax.experimental import pallas as pl
from jax.experimental.pallas import tpu as pltpu
from jax.experimental.pallas import tpu_sc as plsc

NUM_EMB = 1024
DIM = 64
BETA = 0.25

M_TOTAL = 16384
BM = 1024
_CH = 128
_NB = 16
_NW = 32
_ROWS_PER_TILE = M_TOTAL // _NW
_SG_PER_TILE = _ROWS_PER_TILE // DIM


def _vq_block(z_ref, emb_ref, idxsc_ref, loss_ref):
    i = pl.program_id(0)
    znhwc = z_ref[...]
    emb = emb_ref[...]

    a3 = znhwc.reshape(_NB, DIM, DIM)
    zt = jnp.transpose(a3, (0, 2, 1))
    z = zt.reshape(BM, DIM)

    sz = jnp.sum(z * z, axis=1, keepdims=True)
    se = jnp.sum(emb * emb, axis=1)
    mm = jax.lax.dot_general(z, emb, (((1,), (1,)), ((), ())),
                             preferred_element_type=jnp.float32)

    iota = jax.lax.broadcasted_iota(jnp.int32, (_CH, NUM_EMB), 1).astype(
        jnp.float32)
    idx_parts = []
    bsum = None
    for c in range(BM // _CH):
        mm_c = mm[c * _CH:(c + 1) * _CH, :]
        sz_c = sz[c * _CH:(c + 1) * _CH, :]
        d_c = (sz_c + se[None, :]) - 2.0 * mm_c
        dmin_c = jnp.min(d_c, axis=1, keepdims=True)
        cand = jnp.where(d_c == dmin_c, iota, jnp.float32(NUM_EMB))
        idx_parts.append(jnp.min(cand, axis=1))
        part = jnp.sum(dmin_c, axis=(0, 1), keepdims=True)
        bsum = part if bsum is None else bsum + part

    idx_f = jnp.concatenate(idx_parts)
    idxsc_ref[...] = idx_f.astype(jnp.int32)

    @pl.when(i == 0)
    def _init():
        loss_ref[...] = bsum

    @pl.when(i != 0)
    def _acc():
        loss_ref[...] += bsum


@functools.cache
def _make_sc_gather():
    mesh = plsc.VectorSubcoreMesh(core_axis_name="c", subcore_axis_name="s")
    return functools.partial(
        pl.kernel,
        out_type=jax.ShapeDtypeStruct((M_TOTAL // 2, 2 * DIM), jnp.float32),
        mesh=mesh,
        scratch_types=[
            pltpu.VMEM((_ROWS_PER_TILE,), jnp.int32),
            pltpu.VMEM((DIM * NUM_EMB,), jnp.float32),
            pltpu.VMEM((_ROWS_PER_TILE // 2, 2 * DIM), jnp.float32),
            pltpu.SemaphoreType.DMA,
        ],
        compiler_params=pltpu.CompilerParams(needs_layout_passes=False),
    )(_sc_gather_body)


def _sc_gather_body(embt_hbm, idxsc_hbm, out_hbm, idx_v, embt_v, packed_v, sem):
    wid = lax.axis_index("s") * 2 + lax.axis_index("c")
    base = wid * _ROWS_PER_TILE
    emb_copy = pltpu.async_copy(embt_hbm, embt_v, sem)
    pltpu.sync_copy(idxsc_hbm.at[pl.ds(base, _ROWS_PER_TILE)], idx_v)
    emb_copy.wait()

    out_copies = []
    for s in range(_SG_PER_TILE):
        iq = [idx_v[pl.ds(s * DIM + q * 16, 16)] for q in range(DIM // 16)]

        def _row(j, carry):
            off = (s * DIM + j) * DIM
            row = off // 128
            col = off % 128
            for q in range(DIM // 16):
                vals = plsc.load_gather(embt_v, [iq[q] + j * NUM_EMB])
                packed_v.at[row][pl.ds(col + q * 16, 16)] = vals
            return carry

        lax.fori_loop(0, DIM, _row, 0, unroll=4)

        rows_per_sg = DIM // 2
        out_copies.append(pltpu.async_copy(
            packed_v.at[pl.ds(s * rows_per_sg, rows_per_sg)],
            out_hbm.at[pl.ds(wid * (_ROWS_PER_TILE // 2) + s * rows_per_sg,
                             rows_per_sg)],
            sem))
    for c in out_copies:
        c.wait()


def kernel(z_e, emb):
    z_nhwc = jnp.transpose(z_e, (0, 2, 3, 1)).reshape(M_TOTAL, DIM)
    grid = M_TOTAL // BM
    idx_sc, losssum = pl.pallas_call(
        _vq_block,
        grid=(grid,),
        in_specs=[pl.BlockSpec((BM, DIM), lambda i: (i, 0)),
                  pl.BlockSpec((NUM_EMB, DIM), lambda i: (0, 0))],
        out_specs=[pl.BlockSpec((BM,), lambda i: (i,)),
                   pl.BlockSpec((1, 1), lambda i: (0, 0))],
        out_shape=[jax.ShapeDtypeStruct((M_TOTAL,), jnp.int32),
                   jax.ShapeDtypeStruct((1, 1), jnp.float32)],
    )(z_nhwc, emb)

    embt_flat = jnp.transpose(emb).reshape(DIM * NUM_EMB)
    quant_packed = _make_sc_gather()(embt_flat, idx_sc)

    mean_sq = losssum[0, 0] / jnp.float32(M_TOTAL * DIM)
    loss = mean_sq + BETA * mean_sq
    quantized = jnp.transpose(
        quant_packed.reshape(16, 32, 32, DIM), (0, 3, 1, 2))
    encoding_indices = jnp.transpose(
        idx_sc.reshape(16, _NB, DIM), (0, 2, 1)).reshape(M_TOTAL)[:, None]
    return (loss, quantized, encoding_indices)


# speedup vs baseline: 2.1379x; 1.0749x over previous
import functools

import jax
import jax.numpy as jnp
from jax import lax
from jax.experimental import pallas as pl
from jax.experimental.pallas import tpu as pltpu
from jax.experimental.pallas import tpu_sc as plsc

NUM_EMB = 1024
DIM = 64
BETA = 0.25

M_TOTAL = 16384
BM = 2048
_CH = 128
_NB = 16
_NW = 32
_ROWS_PER_TILE = M_TOTAL // _NW
_SG_PER_TILE = _ROWS_PER_TILE // DIM


def _vq_block(z_ref, emb_ref, idxsc_ref, loss_ref):
    i = pl.program_id(0)
    znhwc = z_ref[...]
    emb = emb_ref[...]

    a3 = znhwc.reshape(BM // DIM, DIM, DIM)
    zt = jnp.transpose(a3, (0, 2, 1))
    z = zt.reshape(BM, DIM)

    sz = jnp.sum(z * z, axis=1, keepdims=True)
    se = jnp.sum(emb * emb, axis=1)
    mm = jax.lax.dot_general(z, emb, (((1,), (1,)), ((), ())),
                             preferred_element_type=jnp.float32)

    iota = jax.lax.broadcasted_iota(jnp.int32, (_CH, NUM_EMB), 1).astype(
        jnp.float32)
    idx_parts = []
    bsum = None
    for c in range(BM // _CH):
        mm_c = mm[c * _CH:(c + 1) * _CH, :]
        sz_c = sz[c * _CH:(c + 1) * _CH, :]
        d_c = (sz_c + se[None, :]) - 2.0 * mm_c
        dmin_c = jnp.min(d_c, axis=1, keepdims=True)
        cand = jnp.where(d_c == dmin_c, iota, jnp.float32(NUM_EMB))
        idx_parts.append(jnp.min(cand, axis=1))
        part = jnp.sum(dmin_c, axis=(0, 1), keepdims=True)
        bsum = part if bsum is None else bsum + part

    idx_f = jnp.concatenate(idx_parts)
    idxsc_ref[...] = idx_f.astype(jnp.int32)

    @pl.when(i == 0)
    def _init():
        loss_ref[...] = bsum

    @pl.when(i != 0)
    def _acc():
        loss_ref[...] += bsum


@functools.cache
def _make_sc_gather():
    mesh = plsc.VectorSubcoreMesh(core_axis_name="c", subcore_axis_name="s")
    return functools.partial(
        pl.kernel,
        out_type=jax.ShapeDtypeStruct((M_TOTAL // 2, 2 * DIM), jnp.float32),
        mesh=mesh,
        scratch_types=[
            pltpu.VMEM((_ROWS_PER_TILE,), jnp.int32),
            pltpu.VMEM((DIM * NUM_EMB,), jnp.float32),
            pltpu.VMEM((_ROWS_PER_TILE // 2, 2 * DIM), jnp.float32),
            pltpu.SemaphoreType.DMA,
        ],
        compiler_params=pltpu.CompilerParams(needs_layout_passes=False),
    )(_sc_gather_body)


def _sc_gather_body(embt_hbm, idxsc_hbm, out_hbm, idx_v, embt_v, packed_v, sem):
    wid = lax.axis_index("s") * 2 + lax.axis_index("c")
    base = wid * _ROWS_PER_TILE
    emb_copy = pltpu.async_copy(embt_hbm, embt_v, sem)
    pltpu.sync_copy(idxsc_hbm.at[pl.ds(base, _ROWS_PER_TILE)], idx_v)
    emb_copy.wait()

    out_copies = []
    for s in range(_SG_PER_TILE):
        iq = [idx_v[pl.ds(s * DIM + q * 16, 16)] for q in range(DIM // 16)]

        def _row(j, carry):
            off = (s * DIM + j) * DIM
            row = off // 128
            col = off % 128
            for q in range(DIM // 16):
                vals = plsc.load_gather(embt_v, [iq[q] + j * NUM_EMB])
                packed_v.at[row][pl.ds(col + q * 16, 16)] = vals
            return carry

        lax.fori_loop(0, DIM, _row, 0, unroll=8)

        rows_per_sg = DIM // 2
        out_copies.append(pltpu.async_copy(
            packed_v.at[pl.ds(s * rows_per_sg, rows_per_sg)],
            out_hbm.at[pl.ds(wid * (_ROWS_PER_TILE // 2) + s * rows_per_sg,
                             rows_per_sg)],
            sem))
    for c in out_copies:
        c.wait()


def kernel(z_e, emb):
    z_nhwc = jnp.transpose(z_e, (0, 2, 3, 1)).reshape(M_TOTAL, DIM)
    grid = M_TOTAL // BM
    idx_sc, losssum = pl.pallas_call(
        _vq_block,
        grid=(grid,),
        in_specs=[pl.BlockSpec((BM, DIM), lambda i: (i, 0)),
                  pl.BlockSpec((NUM_EMB, DIM), lambda i: (0, 0))],
        out_specs=[pl.BlockSpec((BM,), lambda i: (i,)),
                   pl.BlockSpec((1, 1), lambda i: (0, 0))],
        out_shape=[jax.ShapeDtypeStruct((M_TOTAL,), jnp.int32),
                   jax.ShapeDtypeStruct((1, 1), jnp.float32)],
    )(z_nhwc, emb)

    embt_flat = jnp.transpose(emb).reshape(DIM * NUM_EMB)
    quant_packed = _make_sc_gather()(embt_flat, idx_sc)

    mean_sq = losssum[0, 0] / jnp.float32(M_TOTAL * DIM)
    loss = mean_sq + BETA * mean_sq
    quantized = jnp.transpose(
        quant_packed.reshape(16, 32, 32, DIM), (0, 3, 1, 2))
    encoding_indices = jnp.transpose(
        idx_sc.reshape(16, _NB, DIM), (0, 2, 1)).reshape(M_TOTAL)[:, None]
    return (loss, quantized, encoding_indices)


# speedup vs baseline: 2.2148x; 1.0360x over previous
import functools

import jax
import jax.numpy as jnp
from jax import lax
from jax.experimental import pallas as pl
from jax.experimental.pallas import tpu as pltpu
from jax.experimental.pallas import tpu_sc as plsc

NUM_EMB = 1024
DIM = 64
BETA = 0.25

M_TOTAL = 16384
BM = 4096
_CH = 128
_NB = 16
_NW = 32
_ROWS_PER_TILE = M_TOTAL // _NW
_SG_PER_TILE = _ROWS_PER_TILE // DIM


def _vq_block(z_ref, emb_ref, idxsc_ref, loss_ref):
    i = pl.program_id(0)
    znhwc = z_ref[...]
    emb = emb_ref[...]

    a3 = znhwc.reshape(BM // DIM, DIM, DIM)
    zt = jnp.transpose(a3, (0, 2, 1))
    z = zt.reshape(BM, DIM)

    sz = jnp.sum(z * z, axis=1, keepdims=True)
    se = jnp.sum(emb * emb, axis=1)
    mm = jax.lax.dot_general(z, emb, (((1,), (1,)), ((), ())),
                             preferred_element_type=jnp.float32)

    iota = jax.lax.broadcasted_iota(jnp.int32, (_CH, NUM_EMB), 1).astype(
        jnp.float32)
    idx_parts = []
    bsum = None
    for c in range(BM // _CH):
        mm_c = mm[c * _CH:(c + 1) * _CH, :]
        sz_c = sz[c * _CH:(c + 1) * _CH, :]
        d_c = (sz_c + se[None, :]) - 2.0 * mm_c
        dmin_c = jnp.min(d_c, axis=1, keepdims=True)
        cand = jnp.where(d_c == dmin_c, iota, jnp.float32(NUM_EMB))
        idx_parts.append(jnp.min(cand, axis=1))
        part = jnp.sum(dmin_c, axis=(0, 1), keepdims=True)
        bsum = part if bsum is None else bsum + part

    idx_f = jnp.concatenate(idx_parts)
    idxsc_ref[...] = idx_f.astype(jnp.int32)

    @pl.when(i == 0)
    def _init():
        loss_ref[...] = bsum

    @pl.when(i != 0)
    def _acc():
        loss_ref[...] += bsum


@functools.cache
def _make_sc_gather():
    mesh = plsc.VectorSubcoreMesh(core_axis_name="c", subcore_axis_name="s")
    return functools.partial(
        pl.kernel,
        out_type=jax.ShapeDtypeStruct((M_TOTAL // 2, 2 * DIM), jnp.float32),
        mesh=mesh,
        scratch_types=[
            pltpu.VMEM((_ROWS_PER_TILE,), jnp.int32),
            pltpu.VMEM((DIM * NUM_EMB,), jnp.float32),
            pltpu.VMEM((_ROWS_PER_TILE // 2, 2 * DIM), jnp.float32),
            pltpu.SemaphoreType.DMA,
        ],
        compiler_params=pltpu.CompilerParams(needs_layout_passes=False),
    )(_sc_gather_body)


def _sc_gather_body(embt_hbm, idxsc_hbm, out_hbm, idx_v, embt_v, packed_v, sem):
    wid = lax.axis_index("s") * 2 + lax.axis_index("c")
    base = wid * _ROWS_PER_TILE
    emb_copy = pltpu.async_copy(embt_hbm, embt_v, sem)
    pltpu.sync_copy(idxsc_hbm.at[pl.ds(base, _ROWS_PER_TILE)], idx_v)
    emb_copy.wait()

    out_copies = []
    for s in range(_SG_PER_TILE):
        iq = [idx_v[pl.ds(s * DIM + q * 16, 16)] for q in range(DIM // 16)]

        def _row(j, carry):
            off = (s * DIM + j) * DIM
            row = off // 128
            col = off % 128
            for q in range(DIM // 16):
                vals = plsc.load_gather(embt_v, [iq[q] + j * NUM_EMB])
                packed_v.at[row][pl.ds(col + q * 16, 16)] = vals
            return carry

        lax.fori_loop(0, DIM, _row, 0, unroll=8)

        rows_per_sg = DIM // 2
        out_copies.append(pltpu.async_copy(
            packed_v.at[pl.ds(s * rows_per_sg, rows_per_sg)],
            out_hbm.at[pl.ds(wid * (_ROWS_PER_TILE // 2) + s * rows_per_sg,
                             rows_per_sg)],
            sem))
    for c in out_copies:
        c.wait()


def kernel(z_e, emb):
    z_nhwc = jnp.transpose(z_e, (0, 2, 3, 1)).reshape(M_TOTAL, DIM)
    grid = M_TOTAL // BM
    idx_sc, losssum = pl.pallas_call(
        _vq_block,
        grid=(grid,),
        in_specs=[pl.BlockSpec((BM, DIM), lambda i: (i, 0)),
                  pl.BlockSpec((NUM_EMB, DIM), lambda i: (0, 0))],
        out_specs=[pl.BlockSpec((BM,), lambda i: (i,)),
                   pl.BlockSpec((1, 1), lambda i: (0, 0))],
        out_shape=[jax.ShapeDtypeStruct((M_TOTAL,), jnp.int32),
                   jax.ShapeDtypeStruct((1, 1), jnp.float32)],
    )(z_nhwc, emb)

    embt_flat = jnp.transpose(emb).reshape(DIM * NUM_EMB)
    quant_packed = _make_sc_gather()(embt_flat, idx_sc)

    mean_sq = losssum[0, 0] / jnp.float32(M_TOTAL * DIM)
    loss = mean_sq + BETA * mean_sq
    quantized = jnp.transpose(
        quant_packed.reshape(16, 32, 32, DIM), (0, 3, 1, 2))
    encoding_indices = jnp.transpose(
        idx_sc.reshape(16, _NB, DIM), (0, 2, 1)).reshape(M_TOTAL)[:, None]
    return (loss, quantized, encoding_indices)
